# Initial kernel scaffold; baseline (speedup 1.0000x reference)
import functools

import jax
import jax.numpy as jnp
from jax import lax
from jax.experimental import pallas as pl
from jax.experimental.pallas import tpu as pltpu
from jax.experimental.pallas import tpu_sc as plsc

_B, _C, _H, _W = 8, 128, 48, 48
_N = _H * _W
_K = 9
_OUT_C = 128
_BLK = 256


def _topk_body(x_ref, idx_ref):
    b = pl.program_id(0)
    i = pl.program_id(1)
    xf = x_ref[...]
    norm = jnp.sqrt(jnp.sum(xf * xf, axis=0, keepdims=True))
    xn = xf / jnp.maximum(norm, 1e-12)
    q = lax.dynamic_slice(xn, (0, i * _BLK), (_C, _BLK))
    sim = lax.dot_general(q, xn, (((0,), (0,)), ((), ())),
                          preferred_element_type=jnp.float32)
    sim = jnp.clip(sim, -1.0, 1.0)
    rows = i * _BLK + lax.broadcasted_iota(jnp.int32, (_BLK, _N), 0)
    cols = lax.broadcasted_iota(jnp.int32, (_BLK, _N), 1)
    sim = jnp.where(cols == rows, 1.1, sim)
    for k in range(_K):
        m = jnp.max(sim, axis=1, keepdims=True)
        idx = jnp.min(jnp.where(sim == m, cols, _N), axis=1, keepdims=True)
        sim = jnp.where(cols == idx, -2.0, sim)
        g = (b * _K + k) * _N + idx[:, 0]
        idx_ref[k, :] = g


def _topk_call(xf):
    return pl.pallas_call(
        _topk_body,
        grid=(_B, _N // _BLK),
        in_specs=[pl.BlockSpec((None, _C, _N), lambda b, i: (b, 0, 0))],
        out_specs=pl.BlockSpec((None, _K, _BLK), lambda b, i: (b, 0, i)),
        out_shape=jax.ShapeDtypeStruct((_B, _K, _N), jnp.int32),
    )(xf)


def _mix_body(x_ref, w_ref, y_ref):
    y_ref[...] = lax.dot_general(
        x_ref[...], w_ref[...], (((0,), (1,)), ((), ())),
        preferred_element_type=jnp.float32)


def _mix_call(xf, wt):
    return pl.pallas_call(
        _mix_body,
        grid=(_B, _K),
        in_specs=[
            pl.BlockSpec((None, _C, _N), lambda b, k: (b, 0, 0)),
            pl.BlockSpec((None, _OUT_C, _C), lambda b, k: (k, 0, 0)),
        ],
        out_specs=pl.BlockSpec((None, None, _N, _OUT_C),
                               lambda b, k: (b, k, 0, 0)),
        out_shape=jax.ShapeDtypeStruct((_B, _K, _N, _OUT_C), jnp.float32),
    )(xf, wt)


_NC, _NS = 2, 16
_NW = _NC * _NS
_T = _N // _NW


def _gather_body(g_hbm, y_hbm, out_hbm, idx_v, buf_v, acc_v, sem):
    wid = lax.axis_index("s") * _NC + lax.axis_index("c")
    tok0 = wid * _T

    def accumulate(r, carry):
        for c in range(_OUT_C // 16):
            sl = pl.ds(c * 16, 16)
            acc_v[r, sl] = acc_v[r, sl] + buf_v[r, sl]
        return carry

    for b in range(_B):
        base = b * _K * _N
        pltpu.sync_copy(g_hbm.at[pl.ds(base + tok0, _T)], idx_v)
        pltpu.async_copy(y_hbm.at[idx_v], acc_v, sem).wait()
        for k in range(1, _K):
            pltpu.sync_copy(g_hbm.at[pl.ds(base + k * _N + tok0, _T)], idx_v)
            pltpu.async_copy(y_hbm.at[idx_v], buf_v, sem).wait()
            lax.fori_loop(0, _T, accumulate, 0)
        pltpu.sync_copy(acc_v, out_hbm.at[pl.ds(b * _N + tok0, _T)])


def _gather_call(gidx, y2d):
    mesh = plsc.VectorSubcoreMesh(core_axis_name="c", subcore_axis_name="s")
    k = functools.partial(
        pl.kernel,
        mesh=mesh,
        out_type=jax.ShapeDtypeStruct((_B * _N, _OUT_C), jnp.float32),
        scratch_types=[
            pltpu.VMEM((_T,), jnp.int32),
            pltpu.VMEM((_T, _OUT_C), jnp.float32),
            pltpu.VMEM((_T, _OUT_C), jnp.float32),
            pltpu.SemaphoreType.DMA,
        ],
    )(_gather_body)
    return k(gidx, y2d)


def kernel(x, W):
    b, c, h, w = x.shape
    n = h * w
    xf = x.reshape(b, c, n)
    wt = jnp.transpose(W, (2, 0, 1))
    gidx = _topk_call(xf).reshape(-1)
    y2d = _mix_call(xf, wt).reshape(_B * _K * _N, _OUT_C)
    out_t = _gather_call(gidx, y2d)
    return out_t.reshape(b, n, _OUT_C).transpose(0, 2, 1).reshape(b, _OUT_C, h, w)


# speedup vs baseline: 27.3152x; 27.3152x over previous
import functools

import jax
import jax.numpy as jnp
from jax import lax
from jax.experimental import pallas as pl
from jax.experimental.pallas import tpu as pltpu
from jax.experimental.pallas import tpu_sc as plsc

_B, _C, _H, _W = 8, 128, 48, 48
_N = _H * _W
_K = 9
_OUT_C = 128
_BLK = 256


def _topk_body(x_ref, idx_ref):
    b = pl.program_id(0)
    i = pl.program_id(1)
    xf = x_ref[...]
    norm = jnp.sqrt(jnp.sum(xf * xf, axis=0, keepdims=True))
    xn = xf / jnp.maximum(norm, 1e-12)
    qr = x_ref[:, pl.ds(i * _BLK, _BLK)]
    qnorm = jnp.sqrt(jnp.sum(qr * qr, axis=0, keepdims=True))
    q = qr / jnp.maximum(qnorm, 1e-12)
    sim = lax.dot_general(q, xn, (((0,), (0,)), ((), ())),
                          preferred_element_type=jnp.float32)
    sim = jnp.clip(sim, -1.0, 1.0)
    rows = i * _BLK + lax.broadcasted_iota(jnp.int32, (_BLK, _N), 0)
    cols = lax.broadcasted_iota(jnp.int32, (_BLK, _N), 1)
    sim = jnp.where(cols == rows, 1.1, sim)
    for k in range(_K):
        m = jnp.max(sim, axis=1, keepdims=True)
        idx = jnp.min(jnp.where(sim == m, cols, _N), axis=1, keepdims=True)
        sim = jnp.where(cols == idx, -2.0, sim)
        g = (b * _K + k) * _N + idx[:, 0]
        idx_ref[k, :] = g


def _topk_call(xf):
    return pl.pallas_call(
        _topk_body,
        grid=(_B, _N // _BLK),
        in_specs=[pl.BlockSpec((None, _C, _N), lambda b, i: (b, 0, 0))],
        out_specs=pl.BlockSpec((None, _K, _BLK), lambda b, i: (b, 0, i)),
        out_shape=jax.ShapeDtypeStruct((_B, _K, _N), jnp.int32),
    )(xf)


def _mix_body(x_ref, w_ref, y_ref):
    y_ref[...] = lax.dot_general(
        x_ref[...], w_ref[...], (((0,), (1,)), ((), ())),
        preferred_element_type=jnp.float32)


def _mix_call(xf, wt):
    return pl.pallas_call(
        _mix_body,
        grid=(_B, _K),
        in_specs=[
            pl.BlockSpec((None, _C, _N), lambda b, k: (b, 0, 0)),
            pl.BlockSpec((None, _OUT_C, _C), lambda b, k: (k, 0, 0)),
        ],
        out_specs=pl.BlockSpec((None, None, _N, _OUT_C),
                               lambda b, k: (b, k, 0, 0)),
        out_shape=jax.ShapeDtypeStruct((_B, _K, _N, _OUT_C), jnp.float32),
    )(xf, wt)


_NC, _NS = 2, 16
_NW = _NC * _NS
_T = _N // _NW


def _gather_body(g_hbm, y_hbm, out_hbm, idx_v, buf_v, acc_v, sem):
    wid = lax.axis_index("s") * _NC + lax.axis_index("c")
    tok0 = wid * _T

    def accumulate(r, carry):
        for c in range(_OUT_C // 16):
            sl = pl.ds(c * 16, 16)
            acc_v[r, sl] = acc_v[r, sl] + buf_v[r, sl]
        return carry

    for b in range(_B):
        base = b * _K * _N
        pltpu.sync_copy(g_hbm.at[pl.ds(base + tok0, _T)], idx_v)
        pltpu.async_copy(y_hbm.at[idx_v], acc_v, sem).wait()
        for k in range(1, _K):
            pltpu.sync_copy(g_hbm.at[pl.ds(base + k * _N + tok0, _T)], idx_v)
            pltpu.async_copy(y_hbm.at[idx_v], buf_v, sem).wait()
            lax.fori_loop(0, _T, accumulate, 0)
        pltpu.sync_copy(acc_v, out_hbm.at[pl.ds(b * _N + tok0, _T)])


def _gather_call(gidx, y2d):
    mesh = plsc.VectorSubcoreMesh(core_axis_name="c", subcore_axis_name="s")
    k = functools.partial(
        pl.kernel,
        mesh=mesh,
        out_type=jax.ShapeDtypeStruct((_B * _N, _OUT_C), jnp.float32),
        scratch_types=[
            pltpu.VMEM((_T,), jnp.int32),
            pltpu.VMEM((_T, _OUT_C), jnp.float32),
            pltpu.VMEM((_T, _OUT_C), jnp.float32),
            pltpu.SemaphoreType.DMA,
        ],
    )(_gather_body)
    return k(gidx, y2d)


def kernel(x, W):
    b, c, h, w = x.shape
    n = h * w
    xf = x.reshape(b, c, n)
    wt = jnp.transpose(W, (2, 0, 1))
    gidx = _topk_call(xf).reshape(-1)
    y2d = _mix_call(xf, wt).reshape(_B * _K * _N, _OUT_C)
    out_t = _gather_call(gidx, y2d)
    return out_t.reshape(b, n, _OUT_C).transpose(0, 2, 1).reshape(b, _OUT_C, h, w)


# speedup vs baseline: 32.8123x; 1.2012x over previous
import functools

import jax
import jax.numpy as jnp
from jax import lax
from jax.experimental import pallas as pl
from jax.experimental.pallas import tpu as pltpu
from jax.experimental.pallas import tpu_sc as plsc

_B, _C, _H, _W = 8, 128, 48, 48
_N = _H * _W
_K = 9
_OUT_C = 128
_BLK = 256


def _topk_body(x_ref, idx_ref):
    b = pl.program_id(0)
    i = pl.program_id(1)
    xf = x_ref[...]
    norm = jnp.sqrt(jnp.sum(xf * xf, axis=0, keepdims=True))
    xn = xf / jnp.maximum(norm, 1e-12)
    qr = x_ref[:, pl.ds(i * _BLK, _BLK)]
    qnorm = jnp.sqrt(jnp.sum(qr * qr, axis=0, keepdims=True))
    q = qr / jnp.maximum(qnorm, 1e-12)
    sim = lax.dot_general(q, xn, (((0,), (0,)), ((), ())),
                          preferred_element_type=jnp.float32)
    sim = jnp.clip(sim, -1.0, 1.0)
    rows = i * _BLK + lax.broadcasted_iota(jnp.int32, (_BLK, _N), 0)
    cols = lax.broadcasted_iota(jnp.int32, (_BLK, _N), 1)
    idx_ref[0, :] = b * _K * _N + rows[:, 0]
    sim = jnp.where(cols == rows, -2.0, sim)
    for k in range(1, _K):
        m = jnp.max(sim, axis=1, keepdims=True)
        idx = jnp.min(jnp.where(sim == m, cols, _N), axis=1, keepdims=True)
        sim = jnp.where(cols == idx, -2.0, sim)
        g = (b * _K + k) * _N + idx[:, 0]
        idx_ref[k, :] = g


def _topk_call(xf):
    return pl.pallas_call(
        _topk_body,
        grid=(_B, _N // _BLK),
        in_specs=[pl.BlockSpec((None, _C, _N), lambda b, i: (b, 0, 0))],
        out_specs=pl.BlockSpec((None, _K, _BLK), lambda b, i: (b, 0, i)),
        out_shape=jax.ShapeDtypeStruct((_B, _K, _N), jnp.int32),
    )(xf)


def _mix_body(x_ref, w_ref, y_ref):
    y_ref[...] = lax.dot_general(
        x_ref[...], w_ref[...], (((0,), (1,)), ((), ())),
        preferred_element_type=jnp.float32)


def _mix_call(xf, wt):
    return pl.pallas_call(
        _mix_body,
        grid=(_B, _K),
        in_specs=[
            pl.BlockSpec((None, _C, _N), lambda b, k: (b, 0, 0)),
            pl.BlockSpec((None, _OUT_C, _C), lambda b, k: (k, 0, 0)),
        ],
        out_specs=pl.BlockSpec((None, None, _N, _OUT_C),
                               lambda b, k: (b, k, 0, 0)),
        out_shape=jax.ShapeDtypeStruct((_B, _K, _N, _OUT_C), jnp.float32),
    )(xf, wt)


_NC, _NS = 2, 16
_NW = _NC * _NS
_T = _N // _NW


def _gather_body(g_hbm, y_hbm, out_hbm, idx_v, buf0, buf1, acc_v,
                 sem_i, sem_a, sem_b0, sem_b1):
    wid = lax.axis_index("s") * _NC + lax.axis_index("c")
    tok0 = wid * _T
    bufs = (buf0, buf1)
    sems = (sem_b0, sem_b1)

    def accumulate(buf):
        def body(r, carry):
            for c in range(_OUT_C // 16):
                sl = pl.ds(c * 16, 16)
                acc_v[r, sl] = acc_v[r, sl] + buf[r, sl]
            return carry
        lax.fori_loop(0, _T, body, 0)

    for b in range(_B):
        base = b * _K * _N
        ih = [pltpu.async_copy(
            g_hbm.at[pl.ds(base + k * _N + tok0, _T)], idx_v.at[k], sem_i)
            for k in range(_K)]
        for h in ih:
            h.wait()
        c_acc = pltpu.async_copy(y_hbm.at[idx_v.at[0]], acc_v, sem_a)
        handles = [None] * (_K + 1)
        handles[1] = pltpu.async_copy(y_hbm.at[idx_v.at[1]], bufs[1], sems[1])
        c_acc.wait()
        for k in range(1, _K):
            if k + 1 < _K:
                nxt = (k + 1) & 1
                handles[k + 1] = pltpu.async_copy(
                    y_hbm.at[idx_v.at[k + 1]], bufs[nxt], sems[nxt])
            handles[k].wait()
            accumulate(bufs[k & 1])
        pltpu.sync_copy(acc_v, out_hbm.at[pl.ds(b * _N + tok0, _T)])


def _gather_call(gidx, y2d):
    mesh = plsc.VectorSubcoreMesh(core_axis_name="c", subcore_axis_name="s")
    k = functools.partial(
        pl.kernel,
        mesh=mesh,
        out_type=jax.ShapeDtypeStruct((_B * _N, _OUT_C), jnp.float32),
        scratch_types=[
            pltpu.VMEM((_K, _T), jnp.int32),
            pltpu.VMEM((_T, _OUT_C), jnp.float32),
            pltpu.VMEM((_T, _OUT_C), jnp.float32),
            pltpu.VMEM((_T, _OUT_C), jnp.float32),
            pltpu.SemaphoreType.DMA,
            pltpu.SemaphoreType.DMA,
            pltpu.SemaphoreType.DMA,
            pltpu.SemaphoreType.DMA,
        ],
    )(_gather_body)
    return k(gidx, y2d)


def kernel(x, W):
    b, c, h, w = x.shape
    n = h * w
    xf = x.reshape(b, c, n)
    wt = jnp.transpose(W, (2, 0, 1))
    gidx = _topk_call(xf).reshape(-1)
    y2d = _mix_call(xf, wt).reshape(_B * _K * _N, _OUT_C)
    out_t = _gather_call(gidx, y2d)
    return out_t.reshape(b, n, _OUT_C).transpose(0, 2, 1).reshape(b, _OUT_C, h, w)


# speedup vs baseline: 40.2844x; 1.2277x over previous
import functools

import jax
import jax.numpy as jnp
from jax import lax
from jax.experimental import pallas as pl
from jax.experimental.pallas import tpu as pltpu
from jax.experimental.pallas import tpu_sc as plsc

_B, _C, _H, _W = 8, 128, 48, 48
_N = _H * _W
_K = 9
_OUT_C = 128
_BLK = 768


def _topk_body(x_ref, idx_ref):
    b = pl.program_id(0)
    i = pl.program_id(1)
    xf = x_ref[...]
    norm = jnp.sqrt(jnp.sum(xf * xf, axis=0, keepdims=True))
    xn = xf / jnp.maximum(norm, 1e-12)
    qr = x_ref[:, pl.ds(i * _BLK, _BLK)]
    qnorm = jnp.sqrt(jnp.sum(qr * qr, axis=0, keepdims=True))
    q = qr / jnp.maximum(qnorm, 1e-12)
    sim = lax.dot_general(q, xn, (((0,), (0,)), ((), ())),
                          preferred_element_type=jnp.float32)
    rowsf = jnp.float32(i * _BLK) + lax.broadcasted_iota(
        jnp.int32, (_BLK, _N), 0).astype(jnp.float32)
    colsf = lax.broadcasted_iota(jnp.int32, (_BLK, _N), 1).astype(jnp.float32)
    rows_i = i * _BLK + lax.broadcasted_iota(jnp.int32, (_BLK, 1), 0)
    idx_ref[0, :] = b * _K * _N + rows_i[:, 0]
    sim = jnp.where(colsf == rowsf, -2.0, sim)
    big = jnp.float32(_N)
    for k in range(1, _K):
        m = jnp.max(sim, axis=1, keepdims=True)
        idxf = jnp.min(jnp.where(sim == m, colsf, big), axis=1, keepdims=True)
        sim = jnp.where(colsf == idxf, -2.0, sim)
        g = (b * _K + k) * _N + idxf[:, 0].astype(jnp.int32)
        idx_ref[k, :] = g


def _topk_call(xf):
    return pl.pallas_call(
        _topk_body,
        grid=(_B, _N // _BLK),
        in_specs=[pl.BlockSpec((None, _C, _N), lambda b, i: (b, 0, 0))],
        out_specs=pl.BlockSpec((None, _K, _BLK), lambda b, i: (b, 0, i)),
        out_shape=jax.ShapeDtypeStruct((_B, _K, _N), jnp.int32),
    )(xf)


def _mix_body(x_ref, w_ref, y_ref):
    y_ref[...] = lax.dot_general(
        x_ref[...], w_ref[...], (((0,), (1,)), ((), ())),
        preferred_element_type=jnp.float32)


def _mix_call(xf, wt):
    return pl.pallas_call(
        _mix_body,
        grid=(_B, _K),
        in_specs=[
            pl.BlockSpec((None, _C, _N), lambda b, k: (b, 0, 0)),
            pl.BlockSpec((None, _OUT_C, _C), lambda b, k: (k, 0, 0)),
        ],
        out_specs=pl.BlockSpec((None, None, _N, _OUT_C),
                               lambda b, k: (b, k, 0, 0)),
        out_shape=jax.ShapeDtypeStruct((_B, _K, _N, _OUT_C), jnp.float32),
    )(xf, wt)


_NC, _NS = 2, 16
_NW = _NC * _NS
_T = _N // _NW


def _gather_body(g_hbm, y_hbm, out_hbm, idx_v, buf0, buf1, acc_v,
                 sem_i, sem_a, sem_b0, sem_b1):
    wid = lax.axis_index("s") * _NC + lax.axis_index("c")
    tok0 = wid * _T
    bufs = (buf0, buf1)
    sems = (sem_b0, sem_b1)

    def accumulate(buf):
        def body(r, carry):
            for c in range(_OUT_C // 16):
                sl = pl.ds(c * 16, 16)
                acc_v[r, sl] = acc_v[r, sl] + buf[r, sl]
            return carry
        lax.fori_loop(0, _T, body, 0)

    for b in range(_B):
        base = b * _K * _N
        ih = [pltpu.async_copy(
            g_hbm.at[pl.ds(base + k * _N + tok0, _T)], idx_v.at[k], sem_i)
            for k in range(_K)]
        for h in ih:
            h.wait()
        c_acc = pltpu.async_copy(y_hbm.at[idx_v.at[0]], acc_v, sem_a)
        handles = [None] * (_K + 1)
        handles[1] = pltpu.async_copy(y_hbm.at[idx_v.at[1]], bufs[1], sems[1])
        c_acc.wait()
        for k in range(1, _K):
            if k + 1 < _K:
                nxt = (k + 1) & 1
                handles[k + 1] = pltpu.async_copy(
                    y_hbm.at[idx_v.at[k + 1]], bufs[nxt], sems[nxt])
            handles[k].wait()
            accumulate(bufs[k & 1])
        pltpu.sync_copy(acc_v, out_hbm.at[pl.ds(b * _N + tok0, _T)])


def _gather_call(gidx, y2d):
    mesh = plsc.VectorSubcoreMesh(core_axis_name="c", subcore_axis_name="s")
    k = functools.partial(
        pl.kernel,
        mesh=mesh,
        out_type=jax.ShapeDtypeStruct((_B * _N, _OUT_C), jnp.float32),
        scratch_types=[
            pltpu.VMEM((_K, _T), jnp.int32),
            pltpu.VMEM((_T, _OUT_C), jnp.float32),
            pltpu.VMEM((_T, _OUT_C), jnp.float32),
            pltpu.VMEM((_T, _OUT_C), jnp.float32),
            pltpu.SemaphoreType.DMA,
            pltpu.SemaphoreType.DMA,
            pltpu.SemaphoreType.DMA,
            pltpu.SemaphoreType.DMA,
        ],
    )(_gather_body)
    return k(gidx, y2d)


def kernel(x, W):
    b, c, h, w = x.shape
    n = h * w
    xf = x.reshape(b, c, n)
    wt = jnp.transpose(W, (2, 0, 1))
    gidx = _topk_call(xf).reshape(-1)
    y2d = _mix_call(xf, wt).reshape(_B * _K * _N, _OUT_C)
    out_t = _gather_call(gidx, y2d)
    return out_t.reshape(b, n, _OUT_C).transpose(0, 2, 1).reshape(b, _OUT_C, h, w)


# speedup vs baseline: 48.2751x; 1.1984x over previous
import functools

import jax
import jax.numpy as jnp
from jax import lax
from jax.experimental import pallas as pl
from jax.experimental.pallas import tpu as pltpu
from jax.experimental.pallas import tpu_sc as plsc

_B, _C, _H, _W = 8, 128, 48, 48
_N = _H * _W
_K = 9
_OUT_C = 128
_BLK = 768


_KPB = _K // (_N // _BLK)


def _topk_body(x_ref, w_ref, idx_ref, y_ref):
    b = pl.program_id(0)
    i = pl.program_id(1)
    xf = x_ref[...]
    norm = jnp.sqrt(jnp.sum(xf * xf, axis=0, keepdims=True))
    xn = xf / jnp.maximum(norm, 1e-12)
    qr = x_ref[:, pl.ds(i * _BLK, _BLK)]
    qnorm = jnp.sqrt(jnp.sum(qr * qr, axis=0, keepdims=True))
    q = qr / jnp.maximum(qnorm, 1e-12)
    sim = lax.dot_general(xn, q, (((0,), (0,)), ((), ())),
                          preferred_element_type=jnp.float32)
    keysf = lax.broadcasted_iota(jnp.int32, (_N, _BLK), 0).astype(jnp.float32)
    qryf = jnp.float32(i * _BLK) + lax.broadcasted_iota(
        jnp.int32, (_N, _BLK), 1).astype(jnp.float32)
    idx_ref[0, :] = (b * _K * _N + i * _BLK
                     + lax.broadcasted_iota(jnp.int32, (1, _BLK), 1)[0, :])
    sim = jnp.where(keysf == qryf, -2.0, sim)
    big = jnp.float32(_N)
    for k in range(1, _K):
        m = jnp.max(sim, axis=0, keepdims=True)
        idxf = jnp.min(jnp.where(sim == m, keysf, big), axis=0, keepdims=True)
        if k + 1 < _K:
            sim = jnp.where(keysf == idxf, -2.0, sim)
        g = (b * _K + k) * _N + idxf[0, :].astype(jnp.int32)
        idx_ref[k, :] = g
    for j in range(_KPB):
        y_ref[j, :, :] = lax.dot_general(
            xf, w_ref[j], (((0,), (1,)), ((), ())),
            preferred_element_type=jnp.float32)


def _topk_call(xf, wt):
    return pl.pallas_call(
        _topk_body,
        grid=(_B, _N // _BLK),
        in_specs=[
            pl.BlockSpec((None, _C, _N), lambda b, i: (b, 0, 0)),
            pl.BlockSpec((_KPB, _OUT_C, _C), lambda b, i: (i, 0, 0)),
        ],
        out_specs=[
            pl.BlockSpec((None, _K, _BLK), lambda b, i: (b, 0, i)),
            pl.BlockSpec((None, _KPB, _N, _OUT_C), lambda b, i: (b, i, 0, 0)),
        ],
        out_shape=[
            jax.ShapeDtypeStruct((_B, _K, _N), jnp.int32),
            jax.ShapeDtypeStruct((_B, _K, _N, _OUT_C), jnp.float32),
        ],
    )(xf, wt)


_NC, _NS = 2, 16
_NW = _NC * _NS
_T = _N // _NW


def _gather_body(g_hbm, y_hbm, out_hbm, idx_v, buf0, buf1, acc_v,
                 sem_i, sem_a, sem_b0, sem_b1):
    wid = lax.axis_index("s") * _NC + lax.axis_index("c")
    tok0 = wid * _T
    bufs = (buf0, buf1)
    sems = (sem_b0, sem_b1)

    def accumulate(buf):
        def body(r, carry):
            for c in range(_OUT_C // 16):
                sl = pl.ds(c * 16, 16)
                acc_v[r, sl] = acc_v[r, sl] + buf[r, sl]
            return carry
        lax.fori_loop(0, _T, body, 0)

    for b in range(_B):
        base = b * _K * _N
        ih = [pltpu.async_copy(
            g_hbm.at[pl.ds(base + k * _N + tok0, _T)], idx_v.at[k], sem_i)
            for k in range(_K)]
        for h in ih:
            h.wait()
        c_acc = pltpu.async_copy(y_hbm.at[idx_v.at[0]], acc_v, sem_a)
        handles = [None] * (_K + 1)
        handles[1] = pltpu.async_copy(y_hbm.at[idx_v.at[1]], bufs[1], sems[1])
        c_acc.wait()
        for k in range(1, _K):
            if k + 1 < _K:
                nxt = (k + 1) & 1
                handles[k + 1] = pltpu.async_copy(
                    y_hbm.at[idx_v.at[k + 1]], bufs[nxt], sems[nxt])
            handles[k].wait()
            accumulate(bufs[k & 1])
        pltpu.sync_copy(acc_v, out_hbm.at[pl.ds(b * _N + tok0, _T)])


def _gather_call(gidx, y2d):
    mesh = plsc.VectorSubcoreMesh(core_axis_name="c", subcore_axis_name="s")
    k = functools.partial(
        pl.kernel,
        mesh=mesh,
        out_type=jax.ShapeDtypeStruct((_B * _N, _OUT_C), jnp.float32),
        scratch_types=[
            pltpu.VMEM((_K, _T), jnp.int32),
            pltpu.VMEM((_T, _OUT_C), jnp.float32),
            pltpu.VMEM((_T, _OUT_C), jnp.float32),
            pltpu.VMEM((_T, _OUT_C), jnp.float32),
            pltpu.SemaphoreType.DMA,
            pltpu.SemaphoreType.DMA,
            pltpu.SemaphoreType.DMA,
            pltpu.SemaphoreType.DMA,
        ],
    )(_gather_body)
    return k(gidx, y2d)


def kernel(x, W):
    b, c, h, w = x.shape
    n = h * w
    xf = x.reshape(b, c, n)
    wt = jnp.transpose(W, (2, 0, 1))
    gidx, y = _topk_call(xf, wt)
    gidx = gidx.reshape(-1)
    y2d = y.reshape(_B * _K * _N, _OUT_C)
    out_t = _gather_call(gidx, y2d)
    return out_t.reshape(b, n, _OUT_C).transpose(0, 2, 1).reshape(b, _OUT_C, h, w)


# speedup vs baseline: 50.8264x; 1.0528x over previous
import functools

import jax
import jax.numpy as jnp
from jax import lax
from jax.experimental import pallas as pl
from jax.experimental.pallas import tpu as pltpu
from jax.experimental.pallas import tpu_sc as plsc

_B, _C, _H, _W = 8, 128, 48, 48
_N = _H * _W
_K = 9
_OUT_C = 128
_BLK = 768


_KPB = _K // (_N // _BLK)


def _topk_body(x_ref, w_ref, idx_ref, y_ref):
    b = pl.program_id(0)
    i = pl.program_id(1)
    xf = x_ref[...]
    norm = jnp.sqrt(jnp.sum(xf * xf, axis=0, keepdims=True))
    xn = xf / jnp.maximum(norm, 1e-12)
    qr = x_ref[:, pl.ds(i * _BLK, _BLK)]
    qnorm = jnp.sqrt(jnp.sum(qr * qr, axis=0, keepdims=True))
    q = qr / jnp.maximum(qnorm, 1e-12)
    sim = lax.dot_general(xn, q, (((0,), (0,)), ((), ())),
                          preferred_element_type=jnp.float32)
    keysf = lax.broadcasted_iota(jnp.int32, (_N, _BLK), 0).astype(jnp.float32)
    qryf = jnp.float32(i * _BLK) + lax.broadcasted_iota(
        jnp.int32, (_N, _BLK), 1).astype(jnp.float32)
    idx_ref[0, :] = (b * _K * _N + i * _BLK
                     + lax.broadcasted_iota(jnp.int32, (1, _BLK), 1)[0, :])
    sim = jnp.where(keysf == qryf, -2.0, sim)
    big = jnp.float32(_N)
    for k in range(1, _K):
        m = jnp.max(sim, axis=0, keepdims=True)
        idxf = jnp.min(jnp.where(sim == m, keysf, big), axis=0, keepdims=True)
        if k + 1 < _K:
            sim = jnp.where(keysf == idxf, -2.0, sim)
        g = (b * _K + k) * _N + idxf[0, :].astype(jnp.int32)
        idx_ref[k, :] = g
    for j in range(_KPB):
        y_ref[j, :, :] = lax.dot_general(
            xf, w_ref[j], (((0,), (1,)), ((), ())),
            preferred_element_type=jnp.float32)


def _topk_call(xf, wt):
    nb = xf.shape[0]
    return pl.pallas_call(
        _topk_body,
        grid=(nb, _N // _BLK),
        in_specs=[
            pl.BlockSpec((None, _C, _N), lambda b, i: (b, 0, 0)),
            pl.BlockSpec((_KPB, _OUT_C, _C), lambda b, i: (i, 0, 0)),
        ],
        out_specs=[
            pl.BlockSpec((None, _K, _BLK), lambda b, i: (b, 0, i)),
            pl.BlockSpec((None, _KPB, _N, _OUT_C), lambda b, i: (b, i, 0, 0)),
        ],
        out_shape=[
            jax.ShapeDtypeStruct((nb, _K, _N), jnp.int32),
            jax.ShapeDtypeStruct((nb, _K, _N, _OUT_C), jnp.float32),
        ],
    )(xf, wt)


_NC, _NS = 2, 16
_NW = _NC * _NS
_T = _N // _NW


def _gather_call(gidx, y2d, nb):
    def _gather_body(g_hbm, y_hbm, out_hbm, idx_v, buf0, buf1, acc_v,
                     sem_i, sem_a, sem_b0, sem_b1):
        wid = lax.axis_index("s") * _NC + lax.axis_index("c")
        tok0 = wid * _T
        bufs = (buf0, buf1)
        sems = (sem_b0, sem_b1)

        def accumulate(buf):
            def body(r, carry):
                for c in range(_OUT_C // 16):
                    sl = pl.ds(c * 16, 16)
                    acc_v[r, sl] = acc_v[r, sl] + buf[r, sl]
                return carry
            lax.fori_loop(0, _T, body, 0)

        for b in range(nb):
            base = b * _K * _N
            ih = [pltpu.async_copy(
                g_hbm.at[pl.ds(base + k * _N + tok0, _T)], idx_v.at[k], sem_i)
                for k in range(_K)]
            for h in ih:
                h.wait()
            c_acc = pltpu.async_copy(y_hbm.at[idx_v.at[0]], acc_v, sem_a)
            handles = [None] * (_K + 1)
            handles[1] = pltpu.async_copy(y_hbm.at[idx_v.at[1]], bufs[1],
                                          sems[1])
            c_acc.wait()
            for k in range(1, _K):
                if k + 1 < _K:
                    nxt = (k + 1) & 1
                    handles[k + 1] = pltpu.async_copy(
                        y_hbm.at[idx_v.at[k + 1]], bufs[nxt], sems[nxt])
                handles[k].wait()
                accumulate(bufs[k & 1])
            pltpu.sync_copy(acc_v, out_hbm.at[pl.ds(b * _N + tok0, _T)])

    mesh = plsc.VectorSubcoreMesh(core_axis_name="c", subcore_axis_name="s")
    k = functools.partial(
        pl.kernel,
        mesh=mesh,
        out_type=jax.ShapeDtypeStruct((nb * _N, _OUT_C), jnp.float32),
        scratch_types=[
            pltpu.VMEM((_K, _T), jnp.int32),
            pltpu.VMEM((_T, _OUT_C), jnp.float32),
            pltpu.VMEM((_T, _OUT_C), jnp.float32),
            pltpu.VMEM((_T, _OUT_C), jnp.float32),
            pltpu.SemaphoreType.DMA,
            pltpu.SemaphoreType.DMA,
            pltpu.SemaphoreType.DMA,
            pltpu.SemaphoreType.DMA,
        ],
    )(_gather_body)
    return k(gidx, y2d)


_G = 2


def kernel(x, W):
    b, c, h, w = x.shape
    n = h * w
    xf = x.reshape(b, c, n)
    wt = jnp.transpose(W, (2, 0, 1))
    bg = b // _G
    outs = []
    for gi in range(_G):
        xfg = lax.slice_in_dim(xf, gi * bg, (gi + 1) * bg, axis=0)
        gidx, y = _topk_call(xfg, wt)
        out_g = _gather_call(gidx.reshape(-1), y.reshape(-1, _OUT_C), bg)
        outs.append(out_g)
    out_t = jnp.concatenate(outs, axis=0)
    return out_t.reshape(b, n, _OUT_C).transpose(0, 2, 1).reshape(b, _OUT_C, h, w)


# speedup vs baseline: 51.3839x; 1.0110x over previous
import functools

import jax
import jax.numpy as jnp
from jax import lax
from jax.experimental import pallas as pl
from jax.experimental.pallas import tpu as pltpu
from jax.experimental.pallas import tpu_sc as plsc

_B, _C, _H, _W = 8, 128, 48, 48
_N = _H * _W
_K = 9
_OUT_C = 128
_BLK = 768


_KPB = _K // (_N // _BLK)


def _topk_body(x_ref, w_ref, idx_ref, y_ref):
    b = pl.program_id(0)
    i = pl.program_id(1)
    xf = x_ref[...]
    norm = jnp.sqrt(jnp.sum(xf * xf, axis=0, keepdims=True))
    xn = xf / jnp.maximum(norm, 1e-12)
    qr = x_ref[:, pl.ds(i * _BLK, _BLK)]
    qnorm = jnp.sqrt(jnp.sum(qr * qr, axis=0, keepdims=True))
    q = qr / jnp.maximum(qnorm, 1e-12)
    sim = lax.dot_general(xn, q, (((0,), (0,)), ((), ())),
                          preferred_element_type=jnp.float32)
    keysf = lax.broadcasted_iota(jnp.int32, (_N, _BLK), 0).astype(jnp.float32)
    qryf = jnp.float32(i * _BLK) + lax.broadcasted_iota(
        jnp.int32, (_N, _BLK), 1).astype(jnp.float32)
    idx_ref[0, :] = (b * _K * _N + i * _BLK
                     + lax.broadcasted_iota(jnp.int32, (1, _BLK), 1)[0, :])
    sim = jnp.where(keysf == qryf, -2.0, sim)
    big = jnp.float32(_N)
    for k in range(1, _K):
        m = jnp.max(sim, axis=0, keepdims=True)
        idxf = jnp.min(jnp.where(sim == m, keysf, big), axis=0, keepdims=True)
        if k + 1 < _K:
            sim = jnp.where(keysf == idxf, -2.0, sim)
        g = (b * _K + k) * _N + idxf[0, :].astype(jnp.int32)
        idx_ref[k, :] = g
    for j in range(_KPB):
        y_ref[j, :, :] = lax.dot_general(
            xf, w_ref[j], (((0,), (1,)), ((), ())),
            preferred_element_type=jnp.float32)


def _topk_call(xf, wt):
    nb = xf.shape[0]
    return pl.pallas_call(
        _topk_body,
        grid=(nb, _N // _BLK),
        in_specs=[
            pl.BlockSpec((None, _C, _N), lambda b, i: (b, 0, 0)),
            pl.BlockSpec((_KPB, _OUT_C, _C), lambda b, i: (i, 0, 0)),
        ],
        out_specs=[
            pl.BlockSpec((None, _K, _BLK), lambda b, i: (b, 0, i)),
            pl.BlockSpec((None, _KPB, _N, _OUT_C), lambda b, i: (b, i, 0, 0)),
        ],
        out_shape=[
            jax.ShapeDtypeStruct((nb, _K, _N), jnp.int32),
            jax.ShapeDtypeStruct((nb, _K, _N, _OUT_C), jnp.float32),
        ],
    )(xf, wt)


_NC, _NS = 2, 16
_NW = _NC * _NS
_T = _N // _NW


def _gather_call(gidx, y2d, nb):
    def _gather_body(g_hbm, y_hbm, out_hbm, idx_v, buf0, buf1, acc_v,
                     sem_i, sem_a, sem_b0, sem_b1):
        wid = lax.axis_index("s") * _NC + lax.axis_index("c")
        tok0 = wid * _T
        bufs = (buf0, buf1)
        sems = (sem_b0, sem_b1)

        def accumulate(buf):
            def body(r, carry):
                for c in range(_OUT_C // 16):
                    sl = pl.ds(c * 16, 16)
                    acc_v[r, sl] = acc_v[r, sl] + buf[r, sl]
                return carry
            lax.fori_loop(0, _T, body, 0)

        for b in range(nb):
            base = b * _K * _N
            ih = [pltpu.async_copy(
                g_hbm.at[pl.ds(base + k * _N + tok0, _T)], idx_v.at[k], sem_i)
                for k in range(_K)]
            for h in ih:
                h.wait()
            c_acc = pltpu.async_copy(y_hbm.at[idx_v.at[0]], acc_v, sem_a)
            handles = [None] * (_K + 1)
            handles[1] = pltpu.async_copy(y_hbm.at[idx_v.at[1]], bufs[1],
                                          sems[1])
            c_acc.wait()
            for k in range(1, _K):
                if k + 1 < _K:
                    nxt = (k + 1) & 1
                    handles[k + 1] = pltpu.async_copy(
                        y_hbm.at[idx_v.at[k + 1]], bufs[nxt], sems[nxt])
                handles[k].wait()
                accumulate(bufs[k & 1])
            pltpu.sync_copy(acc_v, out_hbm.at[pl.ds(b * _N + tok0, _T)])

    mesh = plsc.VectorSubcoreMesh(core_axis_name="c", subcore_axis_name="s")
    k = functools.partial(
        pl.kernel,
        mesh=mesh,
        out_type=jax.ShapeDtypeStruct((nb * _N, _OUT_C), jnp.float32),
        scratch_types=[
            pltpu.VMEM((_K, _T), jnp.int32),
            pltpu.VMEM((_T, _OUT_C), jnp.float32),
            pltpu.VMEM((_T, _OUT_C), jnp.float32),
            pltpu.VMEM((_T, _OUT_C), jnp.float32),
            pltpu.SemaphoreType.DMA,
            pltpu.SemaphoreType.DMA,
            pltpu.SemaphoreType.DMA,
            pltpu.SemaphoreType.DMA,
        ],
    )(_gather_body)
    return k(gidx, y2d)


_G = 4


def kernel(x, W):
    b, c, h, w = x.shape
    n = h * w
    xf = x.reshape(b, c, n)
    wt = jnp.transpose(W, (2, 0, 1))
    bg = b // _G
    outs = []
    for gi in range(_G):
        xfg = lax.slice_in_dim(xf, gi * bg, (gi + 1) * bg, axis=0)
        gidx, y = _topk_call(xfg, wt)
        out_g = _gather_call(gidx.reshape(-1), y.reshape(-1, _OUT_C), bg)
        outs.append(out_g)
    out_t = jnp.concatenate(outs, axis=0)
    return out_t.reshape(b, n, _OUT_C).transpose(0, 2, 1).reshape(b, _OUT_C, h, w)


# speedup vs baseline: 58.0123x; 1.1290x over previous
import functools

import jax
import jax.numpy as jnp
from jax import lax
from jax.experimental import pallas as pl
from jax.experimental.pallas import tpu as pltpu
from jax.experimental.pallas import tpu_sc as plsc

_B, _C, _H, _W = 8, 128, 48, 48
_N = _H * _W
_K = 9
_OUT_C = 128
_BLK = 768


_KPB = _K // (_N // _BLK)


def _topk_body(x_ref, w_ref, idx_ref, y_ref):
    b = pl.program_id(0)
    i = pl.program_id(1)
    xf = x_ref[...]
    norm = jnp.sqrt(jnp.sum(xf * xf, axis=0, keepdims=True))
    xn = xf / jnp.maximum(norm, 1e-12)
    qr = x_ref[:, pl.ds(i * _BLK, _BLK)]
    qnorm = jnp.sqrt(jnp.sum(qr * qr, axis=0, keepdims=True))
    q = qr / jnp.maximum(qnorm, 1e-12)
    sim = lax.dot_general(xn, q, (((0,), (0,)), ((), ())),
                          preferred_element_type=jnp.float32)
    keysi = lax.broadcasted_iota(jnp.int32, (_N, _BLK), 0)
    qryi = i * _BLK + lax.broadcasted_iota(jnp.int32, (_N, _BLK), 1)
    idx_ref[0, :] = (b * _K * _N + i * _BLK
                     + lax.broadcasted_iota(jnp.int32, (1, _BLK), 1)[0, :])
    sim = jnp.where(keysi == qryi, -2.0, sim)
    for k in range(1, _K):
        idx = jnp.argmax(sim, axis=0).astype(jnp.int32)
        if k + 1 < _K:
            sim = jnp.where(keysi == idx[None, :], -2.0, sim)
        idx_ref[k, :] = (b * _K + k) * _N + idx
    for j in range(_KPB):
        y_ref[j, :, :] = lax.dot_general(
            xf, w_ref[j], (((0,), (1,)), ((), ())),
            preferred_element_type=jnp.float32)


def _topk_call(xf, wt):
    nb = xf.shape[0]
    return pl.pallas_call(
        _topk_body,
        grid=(nb, _N // _BLK),
        in_specs=[
            pl.BlockSpec((None, _C, _N), lambda b, i: (b, 0, 0)),
            pl.BlockSpec((_KPB, _OUT_C, _C), lambda b, i: (i, 0, 0)),
        ],
        out_specs=[
            pl.BlockSpec((None, _K, _BLK), lambda b, i: (b, 0, i)),
            pl.BlockSpec((None, _KPB, _N, _OUT_C), lambda b, i: (b, i, 0, 0)),
        ],
        out_shape=[
            jax.ShapeDtypeStruct((nb, _K, _N), jnp.int32),
            jax.ShapeDtypeStruct((nb, _K, _N, _OUT_C), jnp.float32),
        ],
    )(xf, wt)


_NC, _NS = 2, 16
_NW = _NC * _NS
_T = _N // _NW


def _gather_call(gidx, y2d, nb):
    def _gather_body(g_hbm, y_hbm, out_hbm, idx_v, buf0, buf1, acc_v,
                     sem_i, sem_a, sem_b0, sem_b1):
        wid = lax.axis_index("s") * _NC + lax.axis_index("c")
        tok0 = wid * _T
        bufs = (buf0, buf1)
        sems = (sem_b0, sem_b1)

        def accumulate(buf):
            def body(r, carry):
                for c in range(_OUT_C // 16):
                    sl = pl.ds(c * 16, 16)
                    acc_v[r, sl] = acc_v[r, sl] + buf[r, sl]
                return carry
            lax.fori_loop(0, _T, body, 0)

        for b in range(nb):
            base = b * _K * _N
            ih = [pltpu.async_copy(
                g_hbm.at[pl.ds(base + k * _N + tok0, _T)], idx_v.at[k], sem_i)
                for k in range(_K)]
            for h in ih:
                h.wait()
            c_acc = pltpu.async_copy(y_hbm.at[idx_v.at[0]], acc_v, sem_a)
            handles = [None] * (_K + 1)
            handles[1] = pltpu.async_copy(y_hbm.at[idx_v.at[1]], bufs[1],
                                          sems[1])
            c_acc.wait()
            for k in range(1, _K):
                if k + 1 < _K:
                    nxt = (k + 1) & 1
                    handles[k + 1] = pltpu.async_copy(
                        y_hbm.at[idx_v.at[k + 1]], bufs[nxt], sems[nxt])
                handles[k].wait()
                accumulate(bufs[k & 1])
            pltpu.sync_copy(acc_v, out_hbm.at[pl.ds(b * _N + tok0, _T)])

    mesh = plsc.VectorSubcoreMesh(core_axis_name="c", subcore_axis_name="s")
    k = functools.partial(
        pl.kernel,
        mesh=mesh,
        out_type=jax.ShapeDtypeStruct((nb * _N, _OUT_C), jnp.float32),
        scratch_types=[
            pltpu.VMEM((_K, _T), jnp.int32),
            pltpu.VMEM((_T, _OUT_C), jnp.float32),
            pltpu.VMEM((_T, _OUT_C), jnp.float32),
            pltpu.VMEM((_T, _OUT_C), jnp.float32),
            pltpu.SemaphoreType.DMA,
            pltpu.SemaphoreType.DMA,
            pltpu.SemaphoreType.DMA,
            pltpu.SemaphoreType.DMA,
        ],
    )(_gather_body)
    return k(gidx, y2d)


_G = 4


def kernel(x, W):
    b, c, h, w = x.shape
    n = h * w
    xf = x.reshape(b, c, n)
    wt = jnp.transpose(W, (2, 0, 1))
    bg = b // _G
    outs = []
    for gi in range(_G):
        xfg = lax.slice_in_dim(xf, gi * bg, (gi + 1) * bg, axis=0)
        gidx, y = _topk_call(xfg, wt)
        out_g = _gather_call(gidx.reshape(-1), y.reshape(-1, _OUT_C), bg)
        outs.append(out_g)
    out_t = jnp.concatenate(outs, axis=0)
    return out_t.reshape(b, n, _OUT_C).transpose(0, 2, 1).reshape(b, _OUT_C, h, w)
